# Initial kernel scaffold; baseline (speedup 1.0000x reference)
import jax
import jax.numpy as jnp
from jax.experimental import pallas as pl


def kernel(class_labels, class_weights):
    raise NotImplementedError("write your pallas kernel here")


# speedup vs baseline: 1.3550x; 1.3550x over previous
import functools

import jax
import jax.numpy as jnp
from jax import lax
from jax.experimental import pallas as pl
from jax.experimental.pallas import tpu as pltpu
from jax.experimental.pallas import tpu_sc as plsc

NUM_CLASSES = 1000
INPUT_SIZE = 200
BATCH = 16384

_NC = 2
_NS = 16
_NW = _NC * _NS
_B_PER_W = BATCH // _NW


def _sigmoid_body(w_ref, o_ref):
    x = w_ref[...]
    o_ref[...] = 1.0 / (1.0 + jnp.exp(-x))


_sigmoid_table = pl.pallas_call(
    _sigmoid_body,
    out_shape=jax.ShapeDtypeStruct((NUM_CLASSES, INPUT_SIZE), jnp.float32),
)


def _gather_body(idx_hbm, table_hbm, out_hbm, idx_v, rows_v, sem):
    wid = lax.axis_index("s") * _NC + lax.axis_index("c")
    base = wid * _B_PER_W
    pltpu.sync_copy(idx_hbm.at[pl.ds(base, _B_PER_W)], idx_v)
    pltpu.async_copy(table_hbm.at[idx_v], rows_v, sem).wait()
    pltpu.sync_copy(rows_v, out_hbm.at[pl.ds(base, _B_PER_W)])


@functools.cache
def _gather_rows():
    return pl.kernel(
        _gather_body,
        out_type=jax.ShapeDtypeStruct((BATCH, INPUT_SIZE), jnp.float32),
        mesh=plsc.VectorSubcoreMesh(core_axis_name="c", subcore_axis_name="s"),
        scratch_types=[
            pltpu.VMEM((_B_PER_W,), jnp.int32),
            pltpu.VMEM((_B_PER_W, INPUT_SIZE), jnp.float32),
            pltpu.SemaphoreType.DMA,
        ],
        compiler_params=pltpu.CompilerParams(use_tc_tiling_on_sc=False),
    )


def kernel(class_labels, class_weights):
    sig_table = _sigmoid_table(class_weights)
    return _gather_rows()(class_labels.astype(jnp.int32), sig_table)


# speedup vs baseline: 1.4777x; 1.0906x over previous
import functools

import jax
import jax.numpy as jnp
from jax import lax
from jax.experimental import pallas as pl
from jax.experimental.pallas import tpu as pltpu
from jax.experimental.pallas import tpu_sc as plsc

NUM_CLASSES = 1000
INPUT_SIZE = 200
BATCH = 16384

_NC = 2
_NS = 16
_NW = _NC * _NS
_B_PER_W = BATCH // _NW
_CHUNK = 128
_NCHUNK = _B_PER_W // _CHUNK

_ROWS_HI = 63
_ROWS_LO = 62
_TAIL = INPUT_SIZE - 16


def _sigmoid_rows(wv, nr):
    def row_body(r, carry):
        t = wv[r, pl.ds(_TAIL, 16)]
        t = 1.0 / (1.0 + jnp.exp(-t))
        for j in range(INPUT_SIZE // 16):
            v = wv[r, pl.ds(j * 16, 16)]
            wv[r, pl.ds(j * 16, 16)] = 1.0 / (1.0 + jnp.exp(-v))
        wv[r, pl.ds(_TAIL, 16)] = t
        return carry

    lax.fori_loop(0, nr, row_body, 0)


def _body(idx_hbm, w_hbm, out_hbm, wv, idx_v, ra, rb, spt, sem_a, sem_b):
    c = lax.axis_index("c")
    s = lax.axis_index("s")
    wid = s * _NC + c
    base = wid * _B_PER_W

    @pl.when(s < 8)
    def _():
        r0 = s * _ROWS_HI
        pltpu.sync_copy(w_hbm.at[pl.ds(r0, _ROWS_HI)], wv.at[pl.ds(0, _ROWS_HI)])
        _sigmoid_rows(wv, _ROWS_HI)
        pltpu.sync_copy(wv.at[pl.ds(0, _ROWS_HI)], spt.at[pl.ds(r0, _ROWS_HI)])

    @pl.when(s >= 8)
    def _():
        r0 = 8 * _ROWS_HI + (s - 8) * _ROWS_LO
        pltpu.sync_copy(w_hbm.at[pl.ds(r0, _ROWS_LO)], wv.at[pl.ds(0, _ROWS_LO)])
        _sigmoid_rows(wv, _ROWS_LO)
        pltpu.sync_copy(wv.at[pl.ds(0, _ROWS_LO)], spt.at[pl.ds(r0, _ROWS_LO)])

    plsc.subcore_barrier()

    pltpu.sync_copy(idx_hbm.at[pl.ds(base, _B_PER_W)], idx_v)
    bufs = (ra, rb)
    sems = (sem_a, sem_b)
    copies = [None, None]
    copies[0] = pltpu.async_copy(
        spt.at[idx_v.at[pl.ds(0, _CHUNK)]], bufs[0], sems[0])
    for k in range(1, _NCHUNK + 1):
        if k < _NCHUNK:
            copies[k % 2] = pltpu.async_copy(
                spt.at[idx_v.at[pl.ds(k * _CHUNK, _CHUNK)]],
                bufs[k % 2], sems[k % 2])
        j = (k - 1) % 2
        copies[j].wait()
        pltpu.sync_copy(bufs[j], out_hbm.at[pl.ds(base + (k - 1) * _CHUNK, _CHUNK)])


@functools.cache
def _fused():
    return pl.kernel(
        _body,
        out_type=jax.ShapeDtypeStruct((BATCH, INPUT_SIZE), jnp.float32),
        mesh=plsc.VectorSubcoreMesh(core_axis_name="c", subcore_axis_name="s"),
        scratch_types=[
            pltpu.VMEM((_ROWS_HI, INPUT_SIZE), jnp.float32),
            pltpu.VMEM((_B_PER_W,), jnp.int32),
            pltpu.VMEM((_CHUNK, INPUT_SIZE), jnp.float32),
            pltpu.VMEM((_CHUNK, INPUT_SIZE), jnp.float32),
            pltpu.VMEM_SHARED((NUM_CLASSES, INPUT_SIZE), jnp.float32),
            pltpu.SemaphoreType.DMA,
            pltpu.SemaphoreType.DMA,
        ],
        compiler_params=pltpu.CompilerParams(use_tc_tiling_on_sc=False),
    )


def kernel(class_labels, class_weights):
    return _fused()(class_labels.astype(jnp.int32), class_weights)
